# Initial kernel scaffold; baseline (speedup 1.0000x reference)
import jax
import jax.numpy as jnp
from jax.experimental import pallas as pl


def kernel(p0_continuous, p0_binary, p0_controller, p0_action, p0_jumps, p0_character, p1_continuous, p1_binary, p1_controller, p1_action, p1_jumps, p1_character, stage, action_table, jumps_table, char_table, stage_table):
    raise NotImplementedError("write your pallas kernel here")


# speedup vs baseline: 1.7172x; 1.7172x over previous
import functools

import numpy as np

import jax
import jax.numpy as jnp
from jax import lax
from jax.experimental import pallas as pl
from jax.experimental.pallas import tpu as pltpu
from jax.experimental.pallas import tpu_sc as plsc

B = 16384
NW = 32
ROWS_W = B // NW
HALF = ROWS_W // 2
SUB = 128
L = 16
D_OUT = 148
SG = 16
N_SG = HALF // SG
SG_WORDS = SG * D_OUT

_DENSE = ((12, 0), (3, 12), (13, 15), (12, 72), (3, 84), (13, 87))
_DW = SG * sum(w for w, _ in _DENSE)
_N_DCH = _DW // L
_EMB = ((32, 28), (4, 60), (8, 64), (32, 100), (4, 132), (8, 136), (4, 144))


def _dense_pattern():
    out = []
    for w, off in _DENSE:
        s = np.arange(SG * w)
        out.append((s // w) * D_OUT + off + s % w)
    return np.concatenate(out).astype(np.int32)

_PD_HOST = _dense_pattern()


@functools.partial(
    pl.kernel,
    out_type=jax.ShapeDtypeStruct((B * D_OUT,), jnp.float32),
    mesh=plsc.VectorSubcoreMesh(core_axis_name="c", subcore_axis_name="s"),
    compiler_params=pltpu.CompilerParams(
        use_tc_tiling_on_sc=False, needs_layout_passes=False),
    scratch_types=[
        pltpu.VMEM((28, SUB), jnp.int32),
        pltpu.VMEM((_DW,), jnp.int32),
        pltpu.VMEM((SG, _DW), jnp.float32),
        pltpu.VMEM((HALF, 32), jnp.float32),
        pltpu.VMEM((HALF, 8), jnp.float32),
        pltpu.VMEM((HALF, 8), jnp.float32),
        pltpu.VMEM((HALF, 32), jnp.float32),
        pltpu.VMEM((HALF, 8), jnp.float32),
        pltpu.VMEM((HALF, 8), jnp.float32),
        pltpu.VMEM((HALF, 8), jnp.float32),
        pltpu.VMEM((HALF * D_OUT,), jnp.float32),
        pltpu.SemaphoreType.DMA,
    ],
)
def _encode_sc(p0c, p0b, p0k, p1c, p1b, p1k,
               i_p0a, i_p0j, i_p0c, i_p1a, i_p1j, i_p1c, i_stg,
               t_act, t_jmp, t_chr, t_stg, pd_hbm,
               out_hbm,
               idxv, pdv, sd,
               ea0, ej0, ec0, ea1, ej1, ec1, es,
               outb, sem):
    wid = lax.axis_index("s") * 2 + lax.axis_index("c")
    base = wid * ROWS_W
    irow = wid * 4
    drow = wid * 32

    for f, ih in enumerate((i_p0a, i_p0j, i_p0c, i_p1a, i_p1j, i_p1c, i_stg)):
        pltpu.sync_copy(ih.at[pl.ds(irow, 4)], idxv.at[pl.ds(f * 4, 4)])
    pltpu.sync_copy(pd_hbm, pdv)

    emb = ((t_act, ea0), (t_jmp, ej0), (t_chr, ec0),
           (t_act, ea1), (t_jmp, ej1), (t_chr, ec1), (t_stg, es))
    iota = lax.iota(jnp.int32, L)

    def half_body(h, carry):
        rbase = base + h * HALF
        cps = []
        for f, (tab, dst) in enumerate(emb):
            for j in range(2):
                cps.append(pltpu.async_copy(
                    tab.at[idxv.at[f * 4 + h * 2 + j]],
                    dst.at[pl.ds(j * SUB, SUB), :], sem))
        doff = 0
        for dsrc, (w, _) in zip((p0c, p0b, p0k, p1c, p1b, p1k), _DENSE):
            pltpu.sync_copy(dsrc.at[pl.ds(drow + h * SG, SG), :],
                            sd.at[:, pl.ds(doff, SG * w)])
            doff += SG * w
        for cp in cps:
            cp.wait()

        def dense_sg(g, carry):
            def dense_chunk(u, carry):
                dvec = pdv[pl.ds(u * L, L)] + g * SG_WORDS
                vals = sd[g, pl.ds(u * L, L)]
                plsc.store_scatter(outb, [dvec], vals)
                return carry
            return lax.fori_loop(0, _N_DCH, dense_chunk, carry, unroll=4)

        lax.fori_loop(0, N_SG, dense_sg, 0)

        for (w, off), (_, sref) in zip(_EMB, emb):
            lw = w.bit_length() - 1

            def emb_chunk(k, carry, w=w, off=off, sref=sref, lw=lw):
                svec = k * L + iota
                rvec = lax.shift_right_logical(svec, lw)
                cvec = lax.bitwise_and(svec, w - 1)
                vals = plsc.load_gather(sref, [rvec, cvec])
                plsc.store_scatter(outb, [rvec * D_OUT + (cvec + off)], vals)
                return carry

            lax.fori_loop(0, HALF * w // L, emb_chunk, 0, unroll=4)

        pltpu.sync_copy(outb, out_hbm.at[pl.ds(rbase * D_OUT, HALF * D_OUT)])
        return carry

    lax.fori_loop(0, 2, half_body, 0)


def kernel(p0_continuous, p0_binary, p0_controller, p0_action, p0_jumps,
           p0_character, p1_continuous, p1_binary, p1_controller, p1_action,
           p1_jumps, p1_character, stage, action_table, jumps_table,
           char_table, stage_table):
    def idx(a):
        return a.astype(jnp.int32).reshape(B // SUB, SUB)

    def dense(a, w):
        return a.reshape(B // SG, SG * w)
    flat = _encode_sc(
        dense(p0_continuous, 12), dense(p0_binary, 3),
        dense(p0_controller, 13), dense(p1_continuous, 12),
        dense(p1_binary, 3), dense(p1_controller, 13),
        idx(p0_action), idx(p0_jumps), idx(p0_character),
        idx(p1_action), idx(p1_jumps), idx(p1_character), idx(stage),
        action_table, jnp.pad(jumps_table, ((0, 0), (0, 4))), char_table,
        jnp.pad(stage_table, ((0, 0), (0, 4))),
        jnp.asarray(_PD_HOST))
    return flat.reshape(B, D_OUT)


# speedup vs baseline: 1.8086x; 1.0533x over previous
import functools

import numpy as np

import jax
import jax.numpy as jnp
from jax import lax
from jax.experimental import pallas as pl
from jax.experimental.pallas import tpu as pltpu
from jax.experimental.pallas import tpu_sc as plsc

B = 16384
NW = 32
ROWS_W = B // NW
HALF = ROWS_W // 2
SUB = 128
L = 16
D_OUT = 148
SG = 16
N_SG = HALF // SG
SG_WORDS = SG * D_OUT

_DENSE = ((12, 0), (3, 12), (13, 15), (12, 72), (3, 84), (13, 87))
_DW = SG * sum(w for w, _ in _DENSE)
_N_DCH = _DW // L
_EMB = ((32, 28), (4, 60), (8, 64), (32, 100), (4, 132), (8, 136), (4, 144))


def _dense_pattern():
    out = []
    for w, off in _DENSE:
        s = np.arange(SG * w)
        out.append((s // w) * D_OUT + off + s % w)
    return np.concatenate(out).astype(np.int32)

_PD_HOST = _dense_pattern()


@functools.partial(
    pl.kernel,
    out_type=jax.ShapeDtypeStruct((B * D_OUT,), jnp.float32),
    mesh=plsc.VectorSubcoreMesh(core_axis_name="c", subcore_axis_name="s"),
    compiler_params=pltpu.CompilerParams(
        use_tc_tiling_on_sc=False, needs_layout_passes=False),
    scratch_types=[
        pltpu.VMEM((28, SUB), jnp.int32),
        pltpu.VMEM((_DW,), jnp.int32),
        pltpu.VMEM((SG, _DW), jnp.float32),
        pltpu.VMEM((HALF, 32), jnp.float32),
        pltpu.VMEM((HALF, 8), jnp.float32),
        pltpu.VMEM((HALF, 8), jnp.float32),
        pltpu.VMEM((HALF, 32), jnp.float32),
        pltpu.VMEM((HALF, 8), jnp.float32),
        pltpu.VMEM((HALF, 8), jnp.float32),
        pltpu.VMEM((HALF, 8), jnp.float32),
        pltpu.VMEM((HALF * D_OUT,), jnp.float32),
        pltpu.SemaphoreType.DMA,
    ],
)
def _encode_sc(p0c, p0b, p0k, p1c, p1b, p1k,
               i_p0a, i_p0j, i_p0c, i_p1a, i_p1j, i_p1c, i_stg,
               t_act, t_jmp, t_chr, t_stg, pd_hbm,
               out_hbm,
               idxv, pdv, sd,
               ea0, ej0, ec0, ea1, ej1, ec1, es,
               outb, sem):
    wid = lax.axis_index("s") * 2 + lax.axis_index("c")
    base = wid * ROWS_W
    irow = wid * 4
    drow = wid * 32

    for f, ih in enumerate((i_p0a, i_p0j, i_p0c, i_p1a, i_p1j, i_p1c, i_stg)):
        pltpu.sync_copy(ih.at[pl.ds(irow, 4)], idxv.at[pl.ds(f * 4, 4)])
    pltpu.sync_copy(pd_hbm, pdv)

    emb = ((t_act, ea0), (t_jmp, ej0), (t_chr, ec0),
           (t_act, ea1), (t_jmp, ej1), (t_chr, ec1), (t_stg, es))
    iota = lax.iota(jnp.int32, L)

    def half_body(h, carry):
        rbase = base + h * HALF
        cps = []
        for f, (tab, dst) in enumerate(emb):
            for j in range(2):
                cps.append(pltpu.async_copy(
                    tab.at[idxv.at[f * 4 + h * 2 + j]],
                    dst.at[pl.ds(j * SUB, SUB), :], sem))
        doff = 0
        for dsrc, (w, _) in zip((p0c, p0b, p0k, p1c, p1b, p1k), _DENSE):
            pltpu.sync_copy(dsrc.at[pl.ds(drow + h * SG, SG), :],
                            sd.at[:, pl.ds(doff, SG * w)])
            doff += SG * w
        for cp in cps:
            cp.wait()

        def dense_sg(g, carry):
            gw = g * SG_WORDS

            @plsc.parallel_loop(0, _N_DCH, unroll=8)
            def dense_chunk(u):
                dvec = pdv[pl.ds(u * L, L)] + gw
                vals = sd[g, pl.ds(u * L, L)]
                plsc.store_scatter(outb, [dvec], vals)
            return carry

        lax.fori_loop(0, N_SG, dense_sg, 0)

        for (w, off), (_, sref) in zip(_EMB, emb):
            lw = w.bit_length() - 1

            @plsc.parallel_loop(0, HALF * w // L, unroll=8)
            def emb_chunk(k, w=w, off=off, sref=sref, lw=lw):
                svec = k * L + iota
                rvec = lax.shift_right_logical(svec, lw)
                cvec = lax.bitwise_and(svec, w - 1)
                vals = plsc.load_gather(sref, [rvec, cvec])
                plsc.store_scatter(outb, [rvec * D_OUT + (cvec + off)], vals)

        pltpu.sync_copy(outb, out_hbm.at[pl.ds(rbase * D_OUT, HALF * D_OUT)])
        return carry

    lax.fori_loop(0, 2, half_body, 0)


def kernel(p0_continuous, p0_binary, p0_controller, p0_action, p0_jumps,
           p0_character, p1_continuous, p1_binary, p1_controller, p1_action,
           p1_jumps, p1_character, stage, action_table, jumps_table,
           char_table, stage_table):
    def idx(a):
        return a.astype(jnp.int32).reshape(B // SUB, SUB)

    def dense(a, w):
        return a.reshape(B // SG, SG * w)
    flat = _encode_sc(
        dense(p0_continuous, 12), dense(p0_binary, 3),
        dense(p0_controller, 13), dense(p1_continuous, 12),
        dense(p1_binary, 3), dense(p1_controller, 13),
        idx(p0_action), idx(p0_jumps), idx(p0_character),
        idx(p1_action), idx(p1_jumps), idx(p1_character), idx(stage),
        action_table, jnp.pad(jumps_table, ((0, 0), (0, 4))), char_table,
        jnp.pad(stage_table, ((0, 0), (0, 4))),
        jnp.asarray(_PD_HOST))
    return flat.reshape(B, D_OUT)


# speedup vs baseline: 1.8432x; 1.0191x over previous
import functools

import numpy as np

import jax
import jax.numpy as jnp
from jax import lax
from jax.experimental import pallas as pl
from jax.experimental.pallas import tpu as pltpu
from jax.experimental.pallas import tpu_sc as plsc

B = 16384
NW = 32
ROWS_W = B // NW
QTR = 128
L = 16
D_OUT = 148
SG = 16
N_SG = QTR // SG
SG_WORDS = SG * D_OUT
Q_WORDS = QTR * D_OUT

_DENSE = ((12, 0), (3, 12), (13, 15), (12, 72), (3, 84), (13, 87))
_DW = SG * sum(w for w, _ in _DENSE)
_N_DCH = _DW // L
_EMB = ((32, 28), (4, 60), (8, 64), (32, 100), (4, 132), (8, 136), (4, 144))


def _dense_pattern():
    out = []
    for w, off in _DENSE:
        s = np.arange(SG * w)
        out.append((s // w) * D_OUT + off + s % w)
    return np.concatenate(out).astype(np.int32)

_PD_HOST = _dense_pattern()

_SET = [
    pltpu.VMEM((N_SG, _DW), jnp.float32),
    pltpu.VMEM((QTR, 32), jnp.float32),
    pltpu.VMEM((QTR, 8), jnp.float32),
    pltpu.VMEM((QTR, 8), jnp.float32),
    pltpu.VMEM((QTR, 32), jnp.float32),
    pltpu.VMEM((QTR, 8), jnp.float32),
    pltpu.VMEM((QTR, 8), jnp.float32),
    pltpu.VMEM((QTR, 8), jnp.float32),
    pltpu.VMEM((Q_WORDS,), jnp.float32),
    pltpu.SemaphoreType.DMA,
    pltpu.SemaphoreType.DMA,
]


@functools.partial(
    pl.kernel,
    out_type=jax.ShapeDtypeStruct((B * D_OUT,), jnp.float32),
    mesh=plsc.VectorSubcoreMesh(core_axis_name="c", subcore_axis_name="s"),
    compiler_params=pltpu.CompilerParams(
        use_tc_tiling_on_sc=False, needs_layout_passes=False),
    scratch_types=[
        pltpu.VMEM((28, 128), jnp.int32),
        pltpu.VMEM((_DW,), jnp.int32),
        pltpu.SemaphoreType.DMA,
    ] + _SET + _SET,
)
def _encode_sc(p0c, p0b, p0k, p1c, p1b, p1k,
               i_p0a, i_p0j, i_p0c, i_p1a, i_p1j, i_p1c, i_stg,
               t_act, t_jmp, t_chr, t_stg, pd_hbm,
               out_hbm,
               idxv, pdv, psem, *sets):
    setA, setB = sets[:11], sets[11:]
    wid = lax.axis_index("s") * 2 + lax.axis_index("c")
    base = wid * ROWS_W
    irow = wid * 4
    drow = wid * 32

    pcps = [pltpu.async_copy(pd_hbm, pdv, psem)]
    for f, ih in enumerate((i_p0a, i_p0j, i_p0c, i_p1a, i_p1j, i_p1c, i_stg)):
        pcps.append(pltpu.async_copy(
            ih.at[pl.ds(irow, 4)], idxv.at[pl.ds(f * 4, 4)], psem))
    for cp in pcps:
        cp.wait()

    tabs = (t_act, t_jmp, t_chr, t_act, t_jmp, t_chr, t_stg)
    dsrcs = (p0c, p0b, p0k, p1c, p1b, p1k)
    iota = lax.iota(jnp.int32, L)

    def fire(q, S):
        sd, gsem = S[0], S[9]
        cps = [pltpu.async_copy(tab.at[idxv.at[f * 4 + q]], S[1 + f], gsem)
               for f, tab in enumerate(tabs)]
        doff = 0
        for dsrc, (w, _) in zip(dsrcs, _DENSE):
            cps.append(pltpu.async_copy(
                dsrc.at[pl.ds(drow + q * N_SG, N_SG), :],
                sd.at[:, pl.ds(doff, SG * w)], gsem))
            doff += SG * w
        return cps

    def assemble(S):
        sd, outb = S[0], S[8]

        def dense_sg(g, carry):
            gw = g * SG_WORDS

            @plsc.parallel_loop(0, _N_DCH, unroll=8)
            def dense_chunk(u):
                dvec = pdv[pl.ds(u * L, L)] + gw
                vals = sd[g, pl.ds(u * L, L)]
                plsc.store_scatter(outb, [dvec], vals)
            return carry

        lax.fori_loop(0, N_SG, dense_sg, 0)

        for f, (w, off) in enumerate(_EMB):
            sref = S[1 + f]
            lw = w.bit_length() - 1

            @plsc.parallel_loop(0, QTR * w // L, unroll=8)
            def emb_chunk(k, w=w, off=off, sref=sref, lw=lw):
                svec = k * L + iota
                rvec = lax.shift_right_logical(svec, lw)
                cvec = lax.bitwise_and(svec, w - 1)
                vals = plsc.load_gather(sref, [rvec, cvec])
                plsc.store_scatter(outb, [rvec * D_OUT + (cvec + off)], vals)

    stage_cps = {0: fire(0, setA)}
    wb = {}
    for q in range(4):
        S = (setA, setB)[q % 2]
        if q + 1 < 4:
            stage_cps[q + 1] = fire(q + 1, (setA, setB)[(q + 1) % 2])
        for cp in stage_cps.pop(q):
            cp.wait()
        if q >= 2:
            wb[q - 2].wait()
        assemble(S)
        wb[q] = pltpu.async_copy(
            S[8], out_hbm.at[pl.ds((base + q * QTR) * D_OUT, Q_WORDS)], S[10])
    wb[2].wait()
    wb[3].wait()


def kernel(p0_continuous, p0_binary, p0_controller, p0_action, p0_jumps,
           p0_character, p1_continuous, p1_binary, p1_controller, p1_action,
           p1_jumps, p1_character, stage, action_table, jumps_table,
           char_table, stage_table):
    def idx(a):
        return a.astype(jnp.int32).reshape(B // 128, 128)

    def dense(a, w):
        return a.reshape(B // SG, SG * w)
    flat = _encode_sc(
        dense(p0_continuous, 12), dense(p0_binary, 3),
        dense(p0_controller, 13), dense(p1_continuous, 12),
        dense(p1_binary, 3), dense(p1_controller, 13),
        idx(p0_action), idx(p0_jumps), idx(p0_character),
        idx(p1_action), idx(p1_jumps), idx(p1_character), idx(stage),
        action_table, jnp.pad(jumps_table, ((0, 0), (0, 4))), char_table,
        jnp.pad(stage_table, ((0, 0), (0, 4))),
        jnp.asarray(_PD_HOST))
    return flat.reshape(B, D_OUT)


# speedup vs baseline: 3.5923x; 1.9490x over previous
import functools

import numpy as np

import jax
import jax.numpy as jnp
from jax import lax
from jax.experimental import pallas as pl
from jax.experimental.pallas import tpu as pltpu
from jax.experimental.pallas import tpu_sc as plsc

B = 16384
NW = 32
ROWS_W = B // NW
QTR = 128
L = 16
D_OUT = 148
SG = 16
N_SG = QTR // SG
SG_WORDS = SG * D_OUT
Q_WORDS = QTR * D_OUT

_DENSE = ((12, 0), (3, 12), (13, 15), (12, 72), (3, 84), (13, 87))
_DW = SG * sum(w for w, _ in _DENSE)
_N_DCH = _DW // L
_EMB = ((0, 32, 28), (1, 4, 60), (2, 8, 64),
        (0, 32, 100), (1, 4, 132), (2, 8, 136), (3, 4, 144))


def _dense_pattern():
    out = []
    for w, off in _DENSE:
        s = np.arange(SG * w)
        out.append((s // w) * D_OUT + off + s % w)
    return np.concatenate(out).astype(np.int32)

_PD_HOST = _dense_pattern()

_SET = [
    pltpu.VMEM((N_SG, _DW), jnp.float32),
    pltpu.VMEM((Q_WORDS,), jnp.float32),
    pltpu.SemaphoreType.DMA,
    pltpu.SemaphoreType.DMA,
]


@functools.partial(
    pl.kernel,
    out_type=jax.ShapeDtypeStruct((B * D_OUT,), jnp.float32),
    mesh=plsc.VectorSubcoreMesh(core_axis_name="c", subcore_axis_name="s"),
    compiler_params=pltpu.CompilerParams(
        use_tc_tiling_on_sc=False, needs_layout_passes=False),
    scratch_types=[
        pltpu.VMEM((28, 128), jnp.int32),
        pltpu.VMEM((_DW,), jnp.int32),
        pltpu.VMEM((400, 32), jnp.float32),
        pltpu.VMEM((8, 8), jnp.float32),
        pltpu.VMEM((33, 8), jnp.float32),
        pltpu.VMEM((33, 8), jnp.float32),
        pltpu.SemaphoreType.DMA,
    ] + _SET + _SET,
)
def _encode_sc(p0c, p0b, p0k, p1c, p1b, p1k,
               i_p0a, i_p0j, i_p0c, i_p1a, i_p1j, i_p1c, i_stg,
               t_act, t_jmp, t_chr, t_stg, pd_hbm,
               out_hbm,
               idxv, pdv, va, vj, vc, vs, psem, *sets):
    setA, setB = sets[:4], sets[4:]
    vtabs = (va, vj, vc, vs)
    wid = lax.axis_index("s") * 2 + lax.axis_index("c")
    base = wid * ROWS_W
    irow = wid * 4
    drow = wid * 32

    pcps = [pltpu.async_copy(pd_hbm, pdv, psem)]
    for src, dst in zip((t_act, t_jmp, t_chr, t_stg), vtabs):
        pcps.append(pltpu.async_copy(src, dst, psem))
    for f, ih in enumerate((i_p0a, i_p0j, i_p0c, i_p1a, i_p1j, i_p1c, i_stg)):
        pcps.append(pltpu.async_copy(
            ih.at[pl.ds(irow, 4)], idxv.at[pl.ds(f * 4, 4)], psem))
    for cp in pcps:
        cp.wait()

    dsrcs = (p0c, p0b, p0k, p1c, p1b, p1k)
    iota = lax.iota(jnp.int32, L)

    def fire(q, S):
        sd, gsem = S[0], S[2]
        cps = []
        doff = 0
        for dsrc, (w, _) in zip(dsrcs, _DENSE):
            cps.append(pltpu.async_copy(
                dsrc.at[pl.ds(drow + q * N_SG, N_SG), :],
                sd.at[:, pl.ds(doff, SG * w)], gsem))
            doff += SG * w
        return cps

    def assemble(q, S):
        sd, outb = S[0], S[1]

        def dense_sg(g, carry):
            gw = g * SG_WORDS

            @plsc.parallel_loop(0, _N_DCH, unroll=8)
            def dense_chunk(u):
                dvec = pdv[pl.ds(u * L, L)] + gw
                vals = sd[g, pl.ds(u * L, L)]
                plsc.store_scatter(outb, [dvec], vals)
            return carry

        lax.fori_loop(0, N_SG, dense_sg, 0)

        for f, (tid, w, off) in enumerate(_EMB):
            tab = vtabs[tid]
            lw = w.bit_length() - 1
            frow = jnp.full((L,), f * 4 + q, jnp.int32)

            @plsc.parallel_loop(0, QTR * w // L, unroll=8)
            def emb_chunk(k, tab=tab, w=w, off=off, lw=lw, frow=frow):
                svec = k * L + iota
                rvec = lax.shift_right_logical(svec, lw)
                cvec = lax.bitwise_and(svec, w - 1)
                ivec = plsc.load_gather(idxv, [frow, rvec])
                vals = plsc.load_gather(tab, [ivec, cvec])
                plsc.store_scatter(outb, [rvec * D_OUT + (cvec + off)], vals)

    stage_cps = {0: fire(0, setA)}
    wb = {}
    for q in range(4):
        S = (setA, setB)[q % 2]
        if q + 1 < 4:
            stage_cps[q + 1] = fire(q + 1, (setA, setB)[(q + 1) % 2])
        for cp in stage_cps.pop(q):
            cp.wait()
        if q >= 2:
            wb[q - 2].wait()
        assemble(q, S)
        wb[q] = pltpu.async_copy(
            S[1], out_hbm.at[pl.ds((base + q * QTR) * D_OUT, Q_WORDS)], S[3])
    wb[2].wait()
    wb[3].wait()


def kernel(p0_continuous, p0_binary, p0_controller, p0_action, p0_jumps,
           p0_character, p1_continuous, p1_binary, p1_controller, p1_action,
           p1_jumps, p1_character, stage, action_table, jumps_table,
           char_table, stage_table):
    def idx(a):
        return a.astype(jnp.int32).reshape(B // 128, 128)

    def dense(a, w):
        return a.reshape(B // SG, SG * w)
    flat = _encode_sc(
        dense(p0_continuous, 12), dense(p0_binary, 3),
        dense(p0_controller, 13), dense(p1_continuous, 12),
        dense(p1_binary, 3), dense(p1_controller, 13),
        idx(p0_action), idx(p0_jumps), idx(p0_character),
        idx(p1_action), idx(p1_jumps), idx(p1_character), idx(stage),
        action_table, jnp.pad(jumps_table, ((0, 0), (0, 4))), char_table,
        jnp.pad(stage_table, ((0, 0), (0, 4))),
        jnp.asarray(_PD_HOST))
    return flat.reshape(B, D_OUT)
